# baseline (speedup 1.0000x reference)
import jax
import jax.numpy as jnp
from jax.experimental import pallas as pl
from jax.experimental.pallas import tpu as pltpu

N = 1024
IN_F = 128
OUT_F = 64
LOG2E = 1.4426950408889634
TILE = 16
NT = N // TILE


def _gat_kernel(x_ref, adj_ref, w_ref, a_ref, out_ref, p_ref):
    h = jnp.dot(x_ref[...], w_ref[...], preferred_element_type=jnp.float32)
    a_vec = a_ref[...]
    f = jnp.dot(h, a_vec[:OUT_F, :], preferred_element_type=jnp.float32)
    g = jnp.dot(h, a_vec[OUT_F:, :], preferred_element_type=jnp.float32)
    fg = f + jnp.max(g)
    mhat = jnp.maximum(fg, 0.2 * fg)
    u = (f - mhat) * LOG2E
    v = (0.2 * f - mhat) * LOG2E
    g_row = g.reshape(1, N) * LOG2E
    g2_row = 0.2 * g_row

    for t in range(NT):
        lo, hi = t * TILE, (t + 1) * TILE
        e2 = jnp.maximum(u[lo:hi, :] + g_row, v[lo:hi, :] + g2_row)
        p_ref[lo:hi, :] = (adj_ref[lo:hi, :] * jnp.exp2(e2)).astype(jnp.bfloat16)

    ones = jnp.ones((N, 1), dtype=jnp.bfloat16)
    h_ext = jnp.concatenate([h.astype(jnp.bfloat16), ones], axis=1)
    o_ext = jnp.dot(p_ref[...], h_ext, preferred_element_type=jnp.float32)
    denom = o_ext[:, OUT_F:]
    o = o_ext[:, :OUT_F] / denom
    hmean = jnp.sum(h, axis=0, keepdims=True) * (1.0 / N)
    o = jnp.where(denom > 0, o, hmean)
    out_ref[...] = jnp.where(o > 0, o, jnp.exp(o) - 1.0)


@jax.jit
def kernel(x, adj, W, a):
    return pl.pallas_call(
        _gat_kernel,
        scratch_shapes=[pltpu.VMEM((N, N), jnp.bfloat16)],
        out_shape=jax.ShapeDtypeStruct((N, OUT_F), jnp.float32),
    )(x, adj, W, a)


# speedup vs baseline: 1.0041x; 1.0041x over previous
import jax
import jax.numpy as jnp
from jax.experimental import pallas as pl
from jax.experimental.pallas import tpu as pltpu

N = 1024
IN_F = 128
OUT_F = 64
LOG2E = 1.4426950408889634
TILE = 16
NT = N // TILE


def _gat_kernel(x_ref, adj_ref, w_ref, a_ref, out_ref, p_ref):
    h = jnp.dot(x_ref[...], w_ref[...], preferred_element_type=jnp.float32)
    a_vec = a_ref[...]
    f = jnp.dot(h, a_vec[:OUT_F, :], preferred_element_type=jnp.float32)
    g = jnp.dot(h, a_vec[OUT_F:, :], preferred_element_type=jnp.float32)
    fg = f + jnp.max(g)
    mhat = jnp.maximum(fg, 0.2 * fg)
    eu = jnp.exp2((fg - mhat) * LOG2E)
    ev = jnp.exp2((0.2 * fg - mhat) * LOG2E)
    gshift = (g - jnp.max(g)).reshape(1, N)
    eg = jnp.exp2(gshift * LOG2E)
    eg2 = jnp.exp2(gshift * (0.2 * LOG2E))

    for t in range(NT):
        lo, hi = t * TILE, (t + 1) * TILE
        w_tile = jnp.maximum(eu[lo:hi, :] * eg, ev[lo:hi, :] * eg2)
        p_ref[lo:hi, :] = adj_ref[lo:hi, :] * w_tile

    ones = jnp.ones((N, 1), dtype=jnp.float32)
    h_ext = jnp.concatenate([h, ones], axis=1)
    o_ext = jnp.dot(p_ref[...], h_ext, preferred_element_type=jnp.float32)
    denom = o_ext[:, OUT_F:]
    o = o_ext[:, :OUT_F] / denom
    hmean = jnp.sum(h, axis=0, keepdims=True) * (1.0 / N)
    o = jnp.where(denom > 0, o, hmean)
    out_ref[...] = jnp.where(o > 0, o, jnp.exp(o) - 1.0)


@jax.jit
def kernel(x, adj, W, a):
    return pl.pallas_call(
        _gat_kernel,
        scratch_shapes=[pltpu.VMEM((N, N), jnp.float32)],
        out_shape=jax.ShapeDtypeStruct((N, OUT_F), jnp.float32),
    )(x, adj, W, a)
